# baseline (speedup 1.0000x reference)
import jax
import jax.numpy as jnp
from jax import lax
from jax.experimental import pallas as pl
from jax.experimental.pallas import tpu as pltpu
from jax.experimental.pallas import tpu_sc as plsc

N = 10000
F = 128
NPAD = 10240
E = 320000
NW = 32
CB = 128
EC = 79
EP = NW * EC * CB
NB = NPAD // NW
CAPW = 10240
R = 1280
GRID = NPAD // R

_mesh = plsc.VectorSubcoreMesh(core_axis_name="c", subcore_axis_name="s")
_f32 = jnp.float32
_i32 = jnp.int32


def _leaky(t):
    return jnp.where(t >= 0, t, 0.01 * t)


def _wid():
    return lax.axis_index("c") * 16 + lax.axis_index("s")


def _part_body(srcR, dstR, wR, sB, dB, wB, cnts,
               sv, dv, wv, stS, stD, stW, cntv, flv, tmp16i):
    wid = _wid()
    pltpu.sync_copy(srcR.at[wid], sv)
    pltpu.sync_copy(dstR.at[wid], dv)
    pltpu.sync_copy(wR.at[wid], wv)

    zero16i = jnp.zeros((16,), _i32)
    zero16f = jnp.zeros((16,), _f32)
    iota16 = lax.iota(_i32, 16)

    def zi(b, carry):
        cntv[b, :] = zero16i
        flv[b, :] = zero16i
        return carry

    lax.fori_loop(0, NW, zi, 0)

    def group(gi, carry):
        s16 = sv[pl.ds(gi * 16, 16)]
        d16 = dv[pl.ds(gi * 16, 16)]
        w16 = wv[pl.ds(gi * 16, 16)]
        b16 = lax.shift_right_logical(d16 * 6554, 21)
        dl16 = d16 - b16 * NB
        for l in range(16):
            b = b16[l]
            pos16 = cntv[b, :]
            pos = pos16[0]
            msk = iota16 == jnp.full((16,), pos, _i32)
            stS[b, :] = jnp.where(msk, jnp.full((16,), s16[l], _i32), stS[b, :])
            stD[b, :] = jnp.where(msk, jnp.full((16,), dl16[l], _i32), stD[b, :])
            stW[b, :] = jnp.where(msk, jnp.full((16,), w16[l], _f32), stW[b, :])

            @pl.when(pos == 15)
            def _():
                fl = flv[b, :][0]
                pltpu.sync_copy(stS.at[b], sB.at[wid, b, pl.ds(fl * 16, 16)])
                pltpu.sync_copy(stD.at[b], dB.at[wid, b, pl.ds(fl * 16, 16)])
                pltpu.sync_copy(stW.at[b], wB.at[wid, b, pl.ds(fl * 16, 16)])
                flv[b, :] = flv[b, :] + 1
                cntv[b, :] = zero16i

            @pl.when(pos < 15)
            def _():
                cntv[b, :] = pos16 + 1
        return carry

    lax.fori_loop(0, EC * CB // 16, group, 0)

    def drain(b, carry):
        pos = cntv[b, :][0]

        @pl.when(pos > 0)
        def _():
            fl = flv[b, :][0]
            keep = iota16 < jnp.full((16,), pos, _i32)
            stW[b, :] = jnp.where(keep, stW[b, :], zero16f)
            pltpu.sync_copy(stS.at[b], sB.at[wid, b, pl.ds(fl * 16, 16)])
            pltpu.sync_copy(stD.at[b], dB.at[wid, b, pl.ds(fl * 16, 16)])
            pltpu.sync_copy(stW.at[b], wB.at[wid, b, pl.ds(fl * 16, 16)])
            flv[b, :] = flv[b, :] + 1

        fl2 = flv[b, :][0]
        npad = (8 - lax.rem(fl2, 8)) & 7
        stS[b, :] = zero16i
        stD[b, :] = zero16i
        stW[b, :] = zero16f

        def padf(i, c2):
            pltpu.sync_copy(stS.at[b], sB.at[wid, b, pl.ds((fl2 + i) * 16, 16)])
            pltpu.sync_copy(stD.at[b], dB.at[wid, b, pl.ds((fl2 + i) * 16, 16)])
            pltpu.sync_copy(stW.at[b], wB.at[wid, b, pl.ds((fl2 + i) * 16, 16)])
            return c2

        lax.fori_loop(0, npad, padf, 0)
        nch = lax.div(fl2 + npad, 8)
        tmp16i[:] = jnp.full((16,), nch, _i32)
        pltpu.sync_copy(tmp16i, cnts.at[wid, b])
        return carry

    lax.fori_loop(0, NW, drain, 0)


_part = pl.kernel(
    _part_body,
    out_type=(
        jax.ShapeDtypeStruct((NW, NW, CAPW), _i32),
        jax.ShapeDtypeStruct((NW, NW, CAPW), _i32),
        jax.ShapeDtypeStruct((NW, NW, CAPW), _f32),
        jax.ShapeDtypeStruct((NW, NW, 16), _i32),
    ),
    mesh=_mesh,
    scratch_types=[
        pltpu.VMEM((EC * CB,), _i32),
        pltpu.VMEM((EC * CB,), _i32),
        pltpu.VMEM((EC * CB,), _f32),
        pltpu.VMEM((NW, 16), _i32),
        pltpu.VMEM((NW, 16), _i32),
        pltpu.VMEM((NW, 16), _f32),
        pltpu.VMEM((NW, 16), _i32),
        pltpu.VMEM((NW, 16), _i32),
        pltpu.VMEM((16,), _i32),
    ],
)


def _deg_body(dB, wB, cnts, out, dlc, wvc, cv, degT):
    oid = _wid()
    zero16 = jnp.zeros((16,), _f32)

    def z(i, carry):
        degT[i, :] = zero16
        return carry

    lax.fori_loop(0, NB, z, 0)

    def seg(w, carry):
        pltpu.sync_copy(cnts.at[w, oid], cv)
        nch = cv[...][0]

        def chunk(j, c2):
            pltpu.sync_copy(dB.at[w, oid, pl.ds(j * CB, CB)], dlc)
            pltpu.sync_copy(wB.at[w, oid, pl.ds(j * CB, CB)], wvc)
            for bg in range(CB // 16):
                d16 = dlc[pl.ds(bg * 16, 16)]
                w16 = wvc[pl.ds(bg * 16, 16)]
                for l in range(16):
                    dloc = d16[l]
                    degT[dloc, :] = degT[dloc, :] + jnp.full((16,), w16[l], _f32)
            return c2

        lax.fori_loop(0, nch, chunk, 0)
        return carry

    lax.fori_loop(0, NW, seg, 0)
    pltpu.sync_copy(degT, out.at[pl.ds(oid * NB, NB)])


_deg = pl.kernel(
    _deg_body,
    out_type=jax.ShapeDtypeStruct((NPAD, 16), _f32),
    mesh=_mesh,
    scratch_types=[
        pltpu.VMEM((CB,), _i32),
        pltpu.VMEM((CB,), _f32),
        pltpu.VMEM((16,), _i32),
        pltpu.VMEM((NB, 16), _f32),
    ],
)


def _hop_body(g, sB, dB, wB, cnts, out, svc, dlc, wvc, cv, rows, accT):
    oid = _wid()
    zero16 = jnp.zeros((16,), _f32)

    def z(i, carry):
        for gg in range(8):
            accT[i, pl.ds(gg * 16, 16)] = zero16
        return carry

    lax.fori_loop(0, NB, z, 0)

    def seg(w, carry):
        pltpu.sync_copy(cnts.at[w, oid], cv)
        nch = cv[...][0]

        def chunk(j, c2):
            pltpu.sync_copy(sB.at[w, oid, pl.ds(j * CB, CB)], svc)
            pltpu.sync_copy(dB.at[w, oid, pl.ds(j * CB, CB)], dlc)
            pltpu.sync_copy(wB.at[w, oid, pl.ds(j * CB, CB)], wvc)
            pltpu.sync_copy(g.at[svc], rows)
            for bg in range(CB // 16):
                d16 = dlc[pl.ds(bg * 16, 16)]
                w16 = wvc[pl.ds(bg * 16, 16)]
                for l in range(16):
                    e = bg * 16 + l
                    dloc = d16[l]
                    n16 = jnp.full((16,), w16[l], _f32)
                    for gg in range(8):
                        sl = pl.ds(gg * 16, 16)
                        accT[dloc, sl] = accT[dloc, sl] + n16 * rows[e, sl]
            return c2

        lax.fori_loop(0, nch, chunk, 0)
        return carry

    lax.fori_loop(0, NW, seg, 0)
    pltpu.sync_copy(accT, out.at[pl.ds(oid * NB, NB)])


_hop = pl.kernel(
    _hop_body,
    out_type=jax.ShapeDtypeStruct((NPAD, F), _f32),
    mesh=_mesh,
    scratch_types=[
        pltpu.VMEM((CB,), _i32),
        pltpu.VMEM((CB,), _i32),
        pltpu.VMEM((CB,), _f32),
        pltpu.VMEM((16,), _i32),
        pltpu.VMEM((CB, F), _f32),
        pltpu.VMEM((NB, F), _f32),
    ],
)


_row = pl.BlockSpec((R, F), lambda i: (i, 0))
_colb = pl.BlockSpec((R, 1), lambda i: (i, 0))


def _full(shp):
    return pl.BlockSpec(shp, lambda i: tuple(0 for _ in shp))


def _dis_body(deg_ref, dis_ref):
    dsum = deg_ref[...]
    dis = jnp.where(dsum > 0, lax.rsqrt(dsum), 0.0)
    dis_ref[...] = dis[:, 0:1]


_dis = pl.pallas_call(
    _dis_body,
    out_shape=jax.ShapeDtypeStruct((NPAD, 1), _f32),
)


def _enc_body(x_ref, W_ref, b_ref, gl_ref, dis_ref, g_ref, acc_ref):
    x = x_ref[...]
    t = jnp.dot(x, W_ref[0], preferred_element_type=_f32) + b_ref[0]
    h1 = _leaky(t)
    t2 = jnp.dot(h1, W_ref[1], preferred_element_type=_f32) + b_ref[1]
    h = _leaky(t2)
    g_ref[...] = dis_ref[...] * h
    acc_ref[...] = jnp.dot(h, gl_ref[...], preferred_element_type=_f32)


_enc = pl.pallas_call(
    _enc_body,
    grid=(GRID,),
    in_specs=[_row, _full((2, F, F)), _full((2, F)), _full((F, F)), _colb],
    out_specs=[_row, _row],
    out_shape=[
        jax.ShapeDtypeStruct((NPAD, F), _f32),
        jax.ShapeDtypeStruct((NPAD, F), _f32),
    ],
)


def _step_body(p_ref, accin_ref, W_ref, dis_ref, g_ref, acc_ref):
    dis = dis_ref[...]
    xk = dis * p_ref[...]
    g_ref[...] = dis * xk
    acc_ref[...] = accin_ref[...] + jnp.dot(
        xk, W_ref[...], preferred_element_type=_f32
    )


_step = pl.pallas_call(
    _step_body,
    grid=(GRID,),
    in_specs=[_row, _row, _full((F, F)), _colb],
    out_specs=[_row, _row],
    out_shape=[
        jax.ShapeDtypeStruct((NPAD, F), _f32),
        jax.ShapeDtypeStruct((NPAD, F), _f32),
    ],
)


def _lend_body(p_ref, accin_ref, W_ref, b_ref, dis_ref, Wn_ref, g_ref, accn_ref):
    dis = dis_ref[...]
    xk = dis * p_ref[...]
    t = accin_ref[...] + jnp.dot(xk, W_ref[...], preferred_element_type=_f32)
    t = t + b_ref[...]
    h = _leaky(t)
    g_ref[...] = dis * h
    accn_ref[...] = jnp.dot(h, Wn_ref[...], preferred_element_type=_f32)


_lend = pl.pallas_call(
    _lend_body,
    grid=(GRID,),
    in_specs=[_row, _row, _full((F, F)), _full((1, F)), _colb, _full((F, F))],
    out_specs=[_row, _row],
    out_shape=[
        jax.ShapeDtypeStruct((NPAD, F), _f32),
        jax.ShapeDtypeStruct((NPAD, F), _f32),
    ],
)


def _fin_body(p_ref, accin_ref, W_ref, b_ref, dis_ref, dW_ref, db_ref, out_ref):
    dis = dis_ref[...]
    xk = dis * p_ref[...]
    t = accin_ref[...] + jnp.dot(xk, W_ref[...], preferred_element_type=_f32)
    t = t + b_ref[...]
    d = _leaky(jnp.dot(t, dW_ref[0], preferred_element_type=_f32) + db_ref[0])
    out_ref[...] = jnp.dot(d, dW_ref[1], preferred_element_type=_f32) + db_ref[1]


_fin = pl.pallas_call(
    _fin_body,
    grid=(GRID,),
    in_specs=[_row, _row, _full((F, F)), _full((1, F)), _colb,
              _full((2, F, F)), _full((2, F))],
    out_specs=_row,
    out_shape=jax.ShapeDtypeStruct((NPAD, F), _f32),
)


def kernel(x, edge_index, edge_weight, enc_W, enc_b, gnn_lins, gnn_bias, dec_W, dec_b):
    ei = edge_index.astype(_i32)
    src = jnp.pad(ei[0], (0, EP - E)).reshape(NW, EC * CB)
    dst = jnp.pad(ei[1], (0, EP - E)).reshape(NW, EC * CB)
    wR = jnp.pad(edge_weight, (0, EP - E)).reshape(NW, EC * CB)
    x_p = jnp.pad(x, ((0, NPAD - N), (0, 0)))

    sB, dB, wB, cnts = _part(src, dst, wR)
    deg = _deg(dB, wB, cnts)
    dis_col = _dis(deg)
    g, acc = _enc(x_p, enc_W, enc_b, gnn_lins[0, 0], dis_col)

    for l in range(2):
        for k in range(1, 4):
            p = _hop(g, sB, dB, wB, cnts)
            if k < 3:
                g, acc = _step(p, acc, gnn_lins[l, k], dis_col)
        if l == 0:
            g, acc = _lend(
                p, acc, gnn_lins[0, 3],
                gnn_bias[0].reshape(1, F), dis_col, gnn_lins[1, 0],
            )
        else:
            out = _fin(
                p, acc, gnn_lins[1, 3],
                gnn_bias[1].reshape(1, F), dis_col, dec_W, dec_b,
            )
    return out[:N]


# speedup vs baseline: 1.0878x; 1.0878x over previous
import jax
import jax.numpy as jnp
from jax import lax
from jax.experimental import pallas as pl
from jax.experimental.pallas import tpu as pltpu
from jax.experimental.pallas import tpu_sc as plsc

N = 10000
F = 128
NPAD = 10240
E = 320000
NW = 32
CB = 128
EC = 79
EP = NW * EC * CB
NB = NPAD // NW
CAPW = 10240
R = 1280
GRID = NPAD // R

_mesh = plsc.VectorSubcoreMesh(core_axis_name="c", subcore_axis_name="s")
_f32 = jnp.float32
_i32 = jnp.int32


def _leaky(t):
    return jnp.where(t >= 0, t, 0.01 * t)


def _wid():
    return lax.axis_index("c") * 16 + lax.axis_index("s")


C3 = 3 * CB
CAP3 = (CAPW // CB) * C3


def _part_body(srcR, dstR, wR, pk, cnts,
               sv, dv, wv, stS, stD, stW, cntv, flv, tmp16i):
    wid = _wid()
    pltpu.sync_copy(srcR.at[wid], sv)
    pltpu.sync_copy(dstR.at[wid], dv)
    pltpu.sync_copy(wR.at[wid], wv)

    zero16i = jnp.zeros((16,), _i32)
    zero16f = jnp.zeros((16,), _f32)
    iota16 = lax.iota(_i32, 16)

    def zi(b, carry):
        cntv[b, :] = zero16i
        flv[b, :] = zero16i
        return carry

    lax.fori_loop(0, NW, zi, 0)

    def _flush(b, fl):
        base = lax.shift_right_logical(fl, 3) * C3 + (fl & 7) * 16
        pltpu.sync_copy(stS.at[b], pk.at[wid, b, pl.ds(base, 16)])
        pltpu.sync_copy(stD.at[b], pk.at[wid, b, pl.ds(base + CB, 16)])
        pltpu.sync_copy(stW.at[b], pk.at[wid, b, pl.ds(base + 2 * CB, 16)])

    def group(gi, carry):
        s16 = sv[pl.ds(gi * 16, 16)]
        d16 = dv[pl.ds(gi * 16, 16)]
        w16 = wv[pl.ds(gi * 16, 16)]
        b16 = lax.shift_right_logical(d16 * 6554, 21)
        dl16 = d16 - b16 * NB
        w16i = lax.bitcast_convert_type(w16, _i32)
        for l in range(16):
            b = b16[l]
            pos16 = cntv[b, :]
            pos = pos16[0]
            msk = iota16 == jnp.full((16,), pos, _i32)
            stS[b, :] = jnp.where(msk, jnp.full((16,), s16[l], _i32), stS[b, :])
            stD[b, :] = jnp.where(msk, jnp.full((16,), dl16[l], _i32), stD[b, :])
            stW[b, :] = jnp.where(msk, jnp.full((16,), w16i[l], _i32), stW[b, :])

            @pl.when(pos == 15)
            def _():
                _flush(b, flv[b, :][0])
                flv[b, :] = flv[b, :] + 1
                cntv[b, :] = zero16i

            @pl.when(pos < 15)
            def _():
                cntv[b, :] = pos16 + 1
        return carry

    lax.fori_loop(0, EC * CB // 16, group, 0)

    def drain(b, carry):
        pos = cntv[b, :][0]

        @pl.when(pos > 0)
        def _():
            keep = iota16 < jnp.full((16,), pos, _i32)
            stW[b, :] = jnp.where(keep, stW[b, :], zero16i)
            _flush(b, flv[b, :][0])
            flv[b, :] = flv[b, :] + 1

        fl2 = flv[b, :][0]
        npad = jnp.where(fl2 == 0, 8, (8 - lax.rem(fl2, 8)) & 7)
        stS[b, :] = zero16i
        stD[b, :] = zero16i
        stW[b, :] = zero16i

        def padf(i, c2):
            _flush(b, fl2 + i)
            return c2

        lax.fori_loop(0, npad, padf, 0)
        nch = lax.shift_right_logical(fl2 + npad, 3)
        tmp16i[:] = jnp.full((16,), nch, _i32)
        pltpu.sync_copy(tmp16i, cnts.at[b, wid])
        return carry

    lax.fori_loop(0, NW, drain, 0)


_part = pl.kernel(
    _part_body,
    out_type=(
        jax.ShapeDtypeStruct((NW, NW, CAP3), _i32),
        jax.ShapeDtypeStruct((NW, NW, 16), _i32),
    ),
    mesh=_mesh,
    scratch_types=[
        pltpu.VMEM((EC * CB,), _i32),
        pltpu.VMEM((EC * CB,), _i32),
        pltpu.VMEM((EC * CB,), _f32),
        pltpu.VMEM((NW, 16), _i32),
        pltpu.VMEM((NW, 16), _i32),
        pltpu.VMEM((NW, 16), _i32),
        pltpu.VMEM((NW, 16), _i32),
        pltpu.VMEM((NW, 16), _i32),
        pltpu.VMEM((16,), _i32),
    ],
)


def _deg_body(pk, cnts, out, pc, cv, degT):
    oid = _wid()
    zero16 = jnp.zeros((16,), _f32)

    def z(i, carry):
        degT[i, :] = zero16
        return carry

    lax.fori_loop(0, NB, z, 0)
    pltpu.sync_copy(cnts.at[oid], cv)

    def seg(w, carry):
        nch = cv[w, :][0]

        def chunk(j, c2):
            pltpu.sync_copy(pk.at[w, oid, pl.ds(j * C3, C3)], pc)

            def grp(bg, c3):
                d16 = pc[pl.ds(CB + bg * 16, 16)]
                w16 = lax.bitcast_convert_type(pc[pl.ds(2 * CB + bg * 16, 16)], _f32)
                for l in range(16):
                    dloc = d16[l]
                    degT[dloc, :] = degT[dloc, :] + jnp.full((16,), w16[l], _f32)
                return c3

            lax.fori_loop(0, CB // 16, grp, 0)
            return c2

        lax.fori_loop(0, nch, chunk, 0)
        return carry

    lax.fori_loop(0, NW, seg, 0)
    pltpu.sync_copy(degT, out.at[pl.ds(oid * NB, NB)])


_deg = pl.kernel(
    _deg_body,
    out_type=jax.ShapeDtypeStruct((NPAD, 16), _f32),
    mesh=_mesh,
    scratch_types=[
        pltpu.VMEM((C3,), _i32),
        pltpu.VMEM((NW, 16), _i32),
        pltpu.VMEM((NB, 16), _f32),
    ],
)


def _hop_body(g, pk, cnts, out, cv,
              pk0, pk1, pk2, rows0, rows1,
              sS0, sS1, sS2, sG0, sG1, accT):
    oid = _wid()
    zero16 = jnp.zeros((16,), _f32)

    def z(i, carry):
        for gg in range(8):
            accT[i, pl.ds(gg * 16, 16)] = zero16
        return carry

    lax.fori_loop(0, NB, z, 0)
    pltpu.sync_copy(cnts.at[oid], cv)

    def nch_of(w):
        return cv[w, :][0]

    def tot_f(w, t):
        return t + nch_of(w)

    total = lax.fori_loop(0, NW, tot_f, 0)

    def nxt(w, j):
        j2 = j + 1
        wrap = j2 >= nch_of(w)
        return jnp.where(wrap, jnp.minimum(w + 1, NW - 1), w), jnp.where(wrap, 0, j2)

    pks = (pk0, pk1, pk2)
    rws = (rows0, rows1)
    sSs = (sS0, sS1, sS2)
    sGs = (sG0, sG1)

    def stage(w, j, pbuf, sem):
        pltpu.async_copy(pk.at[w, oid, pl.ds(j * C3, C3)], pbuf, sem)

    def gather(pbuf, rbuf, sem):
        pltpu.async_copy(g.at[pbuf.at[pl.ds(0, CB)]], rbuf, sem)

    def wait_stage(pbuf, sem):
        pltpu.make_async_copy(pk.at[0, 0, pl.ds(0, C3)], pbuf, sem).wait()

    def wait_gather(rbuf, sem):
        pltpu.make_async_copy(g.at[pl.ds(0, CB)], rbuf, sem).wait()

    def compute(pbuf, rbuf):
        def grp(bg, c3):
            d16 = pbuf[pl.ds(CB + bg * 16, 16)]
            w16 = lax.bitcast_convert_type(pbuf[pl.ds(2 * CB + bg * 16, 16)], _f32)
            for l in range(16):
                e = bg * 16 + l
                dloc = d16[l]
                n16 = jnp.full((16,), w16[l], _f32)
                for gg in range(8):
                    sl = pl.ds(gg * 16, 16)
                    accT[dloc, sl] = accT[dloc, sl] + n16 * rows_cur[e, sl]
            return c3

        rows_cur = rbuf
        lax.fori_loop(0, CB // 16, grp, 0)

    w1, j1 = nxt(0, 0)
    w2, j2 = nxt(w1, j1)
    pltpu.sync_copy(pk.at[0, oid, pl.ds(0, C3)], pk0)

    @pl.when(total > 1)
    def _():
        stage(w1, j1, pk1, sS1)

    gather(pk0, rows0, sG0)

    NT6 = lax.div(total + 5, 6)

    def macro(i6, carry):
        w0c, j0c, w1c, j1c, w2c, j2c = carry
        for u in range(6):
            i = i6 * 6 + u
            pA = pks[u % 3]
            pB = pks[(u + 1) % 3]
            pC = pks[(u + 2) % 3]
            rA = rws[u % 2]
            rB = rws[(u + 1) % 2]

            @pl.when(i + 1 < total)
            def _(pB=pB, rB=rB, u=u, w1c=w1c, j1c=j1c, w2c=w2c, j2c=j2c, i=i):
                wait_stage(pB, sSs[(u + 1) % 3])
                gather(pB, rB, sGs[(u + 1) % 2])

                @pl.when(i + 2 < total)
                def _():
                    stage(w2c, j2c, pks[(u + 2) % 3], sSs[(u + 2) % 3])

            @pl.when(i < total)
            def _(pA=pA, rA=rA, u=u):
                wait_gather(rA, sGs[u % 2])
                compute(pA, rA)

            w0c, j0c = w1c, j1c
            w1c, j1c = w2c, j2c
            w2c, j2c = nxt(w2c, j2c)
        return (w0c, j0c, w1c, j1c, w2c, j2c)

    lax.fori_loop(0, NT6, macro, (0, 0, w1, j1, w2, j2))
    pltpu.sync_copy(accT, out.at[pl.ds(oid * NB, NB)])


_hop = pl.kernel(
    _hop_body,
    out_type=jax.ShapeDtypeStruct((NPAD, F), _f32),
    mesh=_mesh,
    scratch_types=[
        pltpu.VMEM((NW, 16), _i32),
        pltpu.VMEM((C3,), _i32),
        pltpu.VMEM((C3,), _i32),
        pltpu.VMEM((C3,), _i32),
        pltpu.VMEM((CB, F), _f32),
        pltpu.VMEM((CB, F), _f32),
        pltpu.SemaphoreType.DMA,
        pltpu.SemaphoreType.DMA,
        pltpu.SemaphoreType.DMA,
        pltpu.SemaphoreType.DMA,
        pltpu.SemaphoreType.DMA,
        pltpu.VMEM((NB, F), _f32),
    ],
)


_row = pl.BlockSpec((R, F), lambda i: (i, 0))
_colb = pl.BlockSpec((R, 1), lambda i: (i, 0))


def _full(shp):
    return pl.BlockSpec(shp, lambda i: tuple(0 for _ in shp))


def _dis_body(deg_ref, dis_ref):
    dsum = deg_ref[...]
    dis = jnp.where(dsum > 0, lax.rsqrt(dsum), 0.0)
    dis_ref[...] = dis[:, 0:1]


_dis = pl.pallas_call(
    _dis_body,
    out_shape=jax.ShapeDtypeStruct((NPAD, 1), _f32),
)


def _enc_body(x_ref, W_ref, b_ref, gl_ref, dis_ref, g_ref, acc_ref):
    x = x_ref[...]
    t = jnp.dot(x, W_ref[0], preferred_element_type=_f32) + b_ref[0]
    h1 = _leaky(t)
    t2 = jnp.dot(h1, W_ref[1], preferred_element_type=_f32) + b_ref[1]
    h = _leaky(t2)
    g_ref[...] = dis_ref[...] * h
    acc_ref[...] = jnp.dot(h, gl_ref[...], preferred_element_type=_f32)


_enc = pl.pallas_call(
    _enc_body,
    grid=(GRID,),
    in_specs=[_row, _full((2, F, F)), _full((2, F)), _full((F, F)), _colb],
    out_specs=[_row, _row],
    out_shape=[
        jax.ShapeDtypeStruct((NPAD, F), _f32),
        jax.ShapeDtypeStruct((NPAD, F), _f32),
    ],
)


def _step_body(p_ref, accin_ref, W_ref, dis_ref, g_ref, acc_ref):
    dis = dis_ref[...]
    xk = dis * p_ref[...]
    g_ref[...] = dis * xk
    acc_ref[...] = accin_ref[...] + jnp.dot(
        xk, W_ref[...], preferred_element_type=_f32
    )


_step = pl.pallas_call(
    _step_body,
    grid=(GRID,),
    in_specs=[_row, _row, _full((F, F)), _colb],
    out_specs=[_row, _row],
    out_shape=[
        jax.ShapeDtypeStruct((NPAD, F), _f32),
        jax.ShapeDtypeStruct((NPAD, F), _f32),
    ],
)


def _lend_body(p_ref, accin_ref, W_ref, b_ref, dis_ref, Wn_ref, g_ref, accn_ref):
    dis = dis_ref[...]
    xk = dis * p_ref[...]
    t = accin_ref[...] + jnp.dot(xk, W_ref[...], preferred_element_type=_f32)
    t = t + b_ref[...]
    h = _leaky(t)
    g_ref[...] = dis * h
    accn_ref[...] = jnp.dot(h, Wn_ref[...], preferred_element_type=_f32)


_lend = pl.pallas_call(
    _lend_body,
    grid=(GRID,),
    in_specs=[_row, _row, _full((F, F)), _full((1, F)), _colb, _full((F, F))],
    out_specs=[_row, _row],
    out_shape=[
        jax.ShapeDtypeStruct((NPAD, F), _f32),
        jax.ShapeDtypeStruct((NPAD, F), _f32),
    ],
)


def _fin_body(p_ref, accin_ref, W_ref, b_ref, dis_ref, dW_ref, db_ref, out_ref):
    dis = dis_ref[...]
    xk = dis * p_ref[...]
    t = accin_ref[...] + jnp.dot(xk, W_ref[...], preferred_element_type=_f32)
    t = t + b_ref[...]
    d = _leaky(jnp.dot(t, dW_ref[0], preferred_element_type=_f32) + db_ref[0])
    out_ref[...] = jnp.dot(d, dW_ref[1], preferred_element_type=_f32) + db_ref[1]


_fin = pl.pallas_call(
    _fin_body,
    grid=(GRID,),
    in_specs=[_row, _row, _full((F, F)), _full((1, F)), _colb,
              _full((2, F, F)), _full((2, F))],
    out_specs=_row,
    out_shape=jax.ShapeDtypeStruct((NPAD, F), _f32),
)


def kernel(x, edge_index, edge_weight, enc_W, enc_b, gnn_lins, gnn_bias, dec_W, dec_b):
    ei = edge_index.astype(_i32)
    src = jnp.pad(ei[0], (0, EP - E)).reshape(NW, EC * CB)
    dst = jnp.pad(ei[1], (0, EP - E)).reshape(NW, EC * CB)
    wR = jnp.pad(edge_weight, (0, EP - E)).reshape(NW, EC * CB)
    x_p = jnp.pad(x, ((0, NPAD - N), (0, 0)))

    pkb, cnts = _part(src, dst, wR)
    deg = _deg(pkb, cnts)
    dis_col = _dis(deg)
    g, acc = _enc(x_p, enc_W, enc_b, gnn_lins[0, 0], dis_col)

    for l in range(2):
        for k in range(1, 4):
            p = _hop(g, pkb, cnts)
            if k < 3:
                g, acc = _step(p, acc, gnn_lins[l, k], dis_col)
        if l == 0:
            g, acc = _lend(
                p, acc, gnn_lins[0, 3],
                gnn_bias[0].reshape(1, F), dis_col, gnn_lins[1, 0],
            )
        else:
            out = _fin(
                p, acc, gnn_lins[1, 3],
                gnn_bias[1].reshape(1, F), dis_col, dec_W, dec_b,
            )
    return out[:N]


# speedup vs baseline: 1.1058x; 1.0165x over previous
import jax
import jax.numpy as jnp
from jax import lax
from jax.experimental import pallas as pl
from jax.experimental.pallas import tpu as pltpu
from jax.experimental.pallas import tpu_sc as plsc

N = 10000
F = 128
NPAD = 10240
E = 320000
NW = 32
CB = 128
EC = 79
EP = NW * EC * CB
NB = NPAD // NW
CAPW = 10240
R = 1280
GRID = NPAD // R

_mesh = plsc.VectorSubcoreMesh(core_axis_name="c", subcore_axis_name="s")
_f32 = jnp.float32
_i32 = jnp.int32


def _leaky(t):
    return jnp.where(t >= 0, t, 0.01 * t)


def _wid():
    return lax.axis_index("c") * 16 + lax.axis_index("s")


C3 = 3 * CB
CAP3 = (CAPW // CB) * C3


def _part_body(srcR, dstR, wR, pk, cnts,
               sv, dv, wv, stS, stD, stW, cntv, flv, tmp16i):
    wid = _wid()
    pltpu.sync_copy(srcR.at[wid], sv)
    pltpu.sync_copy(dstR.at[wid], dv)
    pltpu.sync_copy(wR.at[wid], wv)

    zero16i = jnp.zeros((16,), _i32)
    zero16f = jnp.zeros((16,), _f32)
    iota16 = lax.iota(_i32, 16)

    def zi(b, carry):
        cntv[b, :] = zero16i
        flv[b, :] = zero16i
        return carry

    lax.fori_loop(0, NW, zi, 0)

    def _flush(b, fl):
        base = lax.shift_right_logical(fl, 3) * C3 + (fl & 7) * 16
        pltpu.sync_copy(stS.at[b], pk.at[wid, b, pl.ds(base, 16)])
        pltpu.sync_copy(stD.at[b], pk.at[wid, b, pl.ds(base + CB, 16)])
        pltpu.sync_copy(stW.at[b], pk.at[wid, b, pl.ds(base + 2 * CB, 16)])

    def group(gi, carry):
        s16 = sv[pl.ds(gi * 16, 16)]
        d16 = dv[pl.ds(gi * 16, 16)]
        w16 = wv[pl.ds(gi * 16, 16)]
        b16 = lax.shift_right_logical(d16 * 6554, 21)
        dl16 = d16 - b16 * NB
        w16i = lax.bitcast_convert_type(w16, _i32)
        for l in range(16):
            b = b16[l]
            pos16 = cntv[b, :]
            pos = pos16[0]
            msk = iota16 == jnp.full((16,), pos, _i32)
            stS[b, :] = jnp.where(msk, jnp.full((16,), s16[l], _i32), stS[b, :])
            stD[b, :] = jnp.where(msk, jnp.full((16,), dl16[l], _i32), stD[b, :])
            stW[b, :] = jnp.where(msk, jnp.full((16,), w16i[l], _i32), stW[b, :])

            @pl.when(pos == 15)
            def _():
                _flush(b, flv[b, :][0])
                flv[b, :] = flv[b, :] + 1
                cntv[b, :] = zero16i

            @pl.when(pos < 15)
            def _():
                cntv[b, :] = pos16 + 1
        return carry

    lax.fori_loop(0, EC * CB // 16, group, 0)

    def drain(b, carry):
        pos = cntv[b, :][0]

        @pl.when(pos > 0)
        def _():
            keep = iota16 < jnp.full((16,), pos, _i32)
            stW[b, :] = jnp.where(keep, stW[b, :], zero16i)
            _flush(b, flv[b, :][0])
            flv[b, :] = flv[b, :] + 1

        fl2 = flv[b, :][0]
        npad = jnp.where(fl2 == 0, 8, (8 - lax.rem(fl2, 8)) & 7)
        stS[b, :] = zero16i
        stD[b, :] = zero16i
        stW[b, :] = zero16i

        def padf(i, c2):
            _flush(b, fl2 + i)
            return c2

        lax.fori_loop(0, npad, padf, 0)
        nch = lax.shift_right_logical(fl2 + npad, 3)
        tmp16i[:] = jnp.full((16,), nch, _i32)
        pltpu.sync_copy(tmp16i, cnts.at[b, wid])
        return carry

    lax.fori_loop(0, NW, drain, 0)


_part = pl.kernel(
    _part_body,
    out_type=(
        jax.ShapeDtypeStruct((NW, NW, CAP3), _i32),
        jax.ShapeDtypeStruct((NW, NW, 16), _i32),
    ),
    mesh=_mesh,
    scratch_types=[
        pltpu.VMEM((EC * CB,), _i32),
        pltpu.VMEM((EC * CB,), _i32),
        pltpu.VMEM((EC * CB,), _f32),
        pltpu.VMEM((NW, 16), _i32),
        pltpu.VMEM((NW, 16), _i32),
        pltpu.VMEM((NW, 16), _i32),
        pltpu.VMEM((NW, 16), _i32),
        pltpu.VMEM((NW, 16), _i32),
        pltpu.VMEM((16,), _i32),
    ],
)


def _deg_body(pk, cnts, out, pc, cv, degT):
    oid = _wid()
    zero16 = jnp.zeros((16,), _f32)

    def z(i, carry):
        degT[i, :] = zero16
        return carry

    lax.fori_loop(0, NB, z, 0)
    pltpu.sync_copy(cnts.at[oid], cv)

    def seg(w, carry):
        nch = cv[w, :][0]

        def chunk(j, c2):
            pltpu.sync_copy(pk.at[w, oid, pl.ds(j * C3, C3)], pc)

            def grp(bg, c3):
                d16 = pc[pl.ds(CB + bg * 16, 16)]
                w16 = lax.bitcast_convert_type(pc[pl.ds(2 * CB + bg * 16, 16)], _f32)
                for l in range(16):
                    dloc = d16[l]
                    degT[dloc, :] = degT[dloc, :] + jnp.full((16,), w16[l], _f32)
                return c3

            lax.fori_loop(0, CB // 16, grp, 0)
            return c2

        lax.fori_loop(0, nch, chunk, 0)
        return carry

    lax.fori_loop(0, NW, seg, 0)
    pltpu.sync_copy(degT, out.at[pl.ds(oid * NB, NB)])


_deg = pl.kernel(
    _deg_body,
    out_type=jax.ShapeDtypeStruct((NPAD, 16), _f32),
    mesh=_mesh,
    scratch_types=[
        pltpu.VMEM((C3,), _i32),
        pltpu.VMEM((NW, 16), _i32),
        pltpu.VMEM((NB, 16), _f32),
    ],
)


def _hop_body(g, pk, cnts, out, cv,
              pk0, pk1, pk2, rows0, rows1,
              sS0, sS1, sS2, sG0, sG1, accT):
    oid = _wid()
    zero16 = jnp.zeros((16,), _f32)

    def z(i, carry):
        for gg in range(8):
            accT[i, pl.ds(gg * 16, 16)] = zero16
        return carry

    lax.fori_loop(0, NB, z, 0)
    pltpu.sync_copy(cnts.at[oid], cv)

    def nch_of(w):
        return cv[w, :][0]

    def tot_f(w, t):
        return t + nch_of(w)

    total = lax.fori_loop(0, NW, tot_f, 0)

    def nxt(w, j):
        j2 = j + 1
        wrap = j2 >= nch_of(w)
        return jnp.where(wrap, jnp.minimum(w + 1, NW - 1), w), jnp.where(wrap, 0, j2)

    pks = (pk0, pk1, pk2)
    rws = (rows0, rows1)
    sSs = (sS0, sS1, sS2)
    sGs = (sG0, sG1)

    def stage(w, j, pbuf, sem):
        pltpu.async_copy(pk.at[w, oid, pl.ds(j * C3, C3)], pbuf, sem)

    def gather(pbuf, rbuf, sem):
        pltpu.async_copy(g.at[pbuf.at[pl.ds(0, CB)]], rbuf, sem)

    def wait_stage(pbuf, sem):
        pltpu.make_async_copy(pk.at[0, 0, pl.ds(0, C3)], pbuf, sem).wait()

    def wait_gather(rbuf, sem):
        pltpu.make_async_copy(g.at[pl.ds(0, CB)], rbuf, sem).wait()

    def compute(pbuf, rbuf):
        def grp(bg, c3):
            d16 = pbuf[pl.ds(CB + bg * 16, 16)]
            w16 = lax.bitcast_convert_type(pbuf[pl.ds(2 * CB + bg * 16, 16)], _f32)
            for l in range(16):
                e = bg * 16 + l
                dloc = d16[l]
                n16 = jnp.full((16,), w16[l], _f32)
                acc = [accT[dloc, pl.ds(gg * 16, 16)] for gg in range(8)]
                vals = [acc[gg] + n16 * rbuf[e, pl.ds(gg * 16, 16)]
                        for gg in range(8)]
                for gg in range(8):
                    accT[dloc, pl.ds(gg * 16, 16)] = vals[gg]
            return c3

        lax.fori_loop(0, CB // 16, grp, 0)

    w1, j1 = nxt(0, 0)
    w2, j2 = nxt(w1, j1)
    pltpu.sync_copy(pk.at[0, oid, pl.ds(0, C3)], pk0)

    @pl.when(total > 1)
    def _():
        stage(w1, j1, pk1, sS1)

    gather(pk0, rows0, sG0)

    NT6 = lax.div(total + 5, 6)

    def macro(i6, carry):
        w0c, j0c, w1c, j1c, w2c, j2c = carry
        for u in range(6):
            i = i6 * 6 + u
            pA = pks[u % 3]
            pB = pks[(u + 1) % 3]
            pC = pks[(u + 2) % 3]
            rA = rws[u % 2]
            rB = rws[(u + 1) % 2]

            @pl.when(i + 1 < total)
            def _(pB=pB, rB=rB, u=u, w1c=w1c, j1c=j1c, w2c=w2c, j2c=j2c, i=i):
                wait_stage(pB, sSs[(u + 1) % 3])
                gather(pB, rB, sGs[(u + 1) % 2])

                @pl.when(i + 2 < total)
                def _():
                    stage(w2c, j2c, pks[(u + 2) % 3], sSs[(u + 2) % 3])

            @pl.when(i < total)
            def _(pA=pA, rA=rA, u=u):
                wait_gather(rA, sGs[u % 2])
                compute(pA, rA)

            w0c, j0c = w1c, j1c
            w1c, j1c = w2c, j2c
            w2c, j2c = nxt(w2c, j2c)
        return (w0c, j0c, w1c, j1c, w2c, j2c)

    lax.fori_loop(0, NT6, macro, (0, 0, w1, j1, w2, j2))
    pltpu.sync_copy(accT, out.at[pl.ds(oid * NB, NB)])


_hop = pl.kernel(
    _hop_body,
    out_type=jax.ShapeDtypeStruct((NPAD, F), _f32),
    mesh=_mesh,
    scratch_types=[
        pltpu.VMEM((NW, 16), _i32),
        pltpu.VMEM((C3,), _i32),
        pltpu.VMEM((C3,), _i32),
        pltpu.VMEM((C3,), _i32),
        pltpu.VMEM((CB, F), _f32),
        pltpu.VMEM((CB, F), _f32),
        pltpu.SemaphoreType.DMA,
        pltpu.SemaphoreType.DMA,
        pltpu.SemaphoreType.DMA,
        pltpu.SemaphoreType.DMA,
        pltpu.SemaphoreType.DMA,
        pltpu.VMEM((NB, F), _f32),
    ],
)


_row = pl.BlockSpec((R, F), lambda i: (i, 0))
_colb = pl.BlockSpec((R, 1), lambda i: (i, 0))


def _full(shp):
    return pl.BlockSpec(shp, lambda i: tuple(0 for _ in shp))


def _dis_body(deg_ref, dis_ref):
    dsum = deg_ref[...]
    dis = jnp.where(dsum > 0, lax.rsqrt(dsum), 0.0)
    dis_ref[...] = dis[:, 0:1]


_dis = pl.pallas_call(
    _dis_body,
    out_shape=jax.ShapeDtypeStruct((NPAD, 1), _f32),
)


def _enc_body(x_ref, W_ref, b_ref, gl_ref, dis_ref, g_ref, acc_ref):
    x = x_ref[...]
    t = jnp.dot(x, W_ref[0], preferred_element_type=_f32) + b_ref[0]
    h1 = _leaky(t)
    t2 = jnp.dot(h1, W_ref[1], preferred_element_type=_f32) + b_ref[1]
    h = _leaky(t2)
    g_ref[...] = dis_ref[...] * h
    acc_ref[...] = jnp.dot(h, gl_ref[...], preferred_element_type=_f32)


_enc = pl.pallas_call(
    _enc_body,
    grid=(GRID,),
    in_specs=[_row, _full((2, F, F)), _full((2, F)), _full((F, F)), _colb],
    out_specs=[_row, _row],
    out_shape=[
        jax.ShapeDtypeStruct((NPAD, F), _f32),
        jax.ShapeDtypeStruct((NPAD, F), _f32),
    ],
)


def _step_body(p_ref, accin_ref, W_ref, dis_ref, g_ref, acc_ref):
    dis = dis_ref[...]
    xk = dis * p_ref[...]
    g_ref[...] = dis * xk
    acc_ref[...] = accin_ref[...] + jnp.dot(
        xk, W_ref[...], preferred_element_type=_f32
    )


_step = pl.pallas_call(
    _step_body,
    grid=(GRID,),
    in_specs=[_row, _row, _full((F, F)), _colb],
    out_specs=[_row, _row],
    out_shape=[
        jax.ShapeDtypeStruct((NPAD, F), _f32),
        jax.ShapeDtypeStruct((NPAD, F), _f32),
    ],
)


def _lend_body(p_ref, accin_ref, W_ref, b_ref, dis_ref, Wn_ref, g_ref, accn_ref):
    dis = dis_ref[...]
    xk = dis * p_ref[...]
    t = accin_ref[...] + jnp.dot(xk, W_ref[...], preferred_element_type=_f32)
    t = t + b_ref[...]
    h = _leaky(t)
    g_ref[...] = dis * h
    accn_ref[...] = jnp.dot(h, Wn_ref[...], preferred_element_type=_f32)


_lend = pl.pallas_call(
    _lend_body,
    grid=(GRID,),
    in_specs=[_row, _row, _full((F, F)), _full((1, F)), _colb, _full((F, F))],
    out_specs=[_row, _row],
    out_shape=[
        jax.ShapeDtypeStruct((NPAD, F), _f32),
        jax.ShapeDtypeStruct((NPAD, F), _f32),
    ],
)


def _fin_body(p_ref, accin_ref, W_ref, b_ref, dis_ref, dW_ref, db_ref, out_ref):
    dis = dis_ref[...]
    xk = dis * p_ref[...]
    t = accin_ref[...] + jnp.dot(xk, W_ref[...], preferred_element_type=_f32)
    t = t + b_ref[...]
    d = _leaky(jnp.dot(t, dW_ref[0], preferred_element_type=_f32) + db_ref[0])
    out_ref[...] = jnp.dot(d, dW_ref[1], preferred_element_type=_f32) + db_ref[1]


_fin = pl.pallas_call(
    _fin_body,
    grid=(GRID,),
    in_specs=[_row, _row, _full((F, F)), _full((1, F)), _colb,
              _full((2, F, F)), _full((2, F))],
    out_specs=_row,
    out_shape=jax.ShapeDtypeStruct((NPAD, F), _f32),
)


def kernel(x, edge_index, edge_weight, enc_W, enc_b, gnn_lins, gnn_bias, dec_W, dec_b):
    ei = edge_index.astype(_i32)
    src = jnp.pad(ei[0], (0, EP - E)).reshape(NW, EC * CB)
    dst = jnp.pad(ei[1], (0, EP - E)).reshape(NW, EC * CB)
    wR = jnp.pad(edge_weight, (0, EP - E)).reshape(NW, EC * CB)
    x_p = jnp.pad(x, ((0, NPAD - N), (0, 0)))

    pkb, cnts = _part(src, dst, wR)
    deg = _deg(pkb, cnts)
    dis_col = _dis(deg)
    g, acc = _enc(x_p, enc_W, enc_b, gnn_lins[0, 0], dis_col)

    for l in range(2):
        for k in range(1, 4):
            p = _hop(g, pkb, cnts)
            if k < 3:
                g, acc = _step(p, acc, gnn_lins[l, k], dis_col)
        if l == 0:
            g, acc = _lend(
                p, acc, gnn_lins[0, 3],
                gnn_bias[0].reshape(1, F), dis_col, gnn_lins[1, 0],
            )
        else:
            out = _fin(
                p, acc, gnn_lins[1, 3],
                gnn_bias[1].reshape(1, F), dis_col, dec_W, dec_b,
            )
    return out[:N]


# speedup vs baseline: 1.1069x; 1.0010x over previous
import jax
import jax.numpy as jnp
from jax import lax
from jax.experimental import pallas as pl
from jax.experimental.pallas import tpu as pltpu
from jax.experimental.pallas import tpu_sc as plsc

N = 10000
F = 128
NPAD = 10240
E = 320000
NW = 32
CB = 128
EC = 79
EP = NW * EC * CB
NB = NPAD // NW
CAPW = 10240
R = 1280
GRID = NPAD // R

_mesh = plsc.VectorSubcoreMesh(core_axis_name="c", subcore_axis_name="s")
_f32 = jnp.float32
_i32 = jnp.int32


def _leaky(t):
    return jnp.where(t >= 0, t, 0.01 * t)


def _wid():
    return lax.axis_index("c") * 16 + lax.axis_index("s")


C3 = 3 * CB
CAP3 = (CAPW // CB) * C3


def _part_body(srcR, dstR, wR, pk, cnts,
               sv, dv, wv, stS, stD, stW, cntv, flv, tmp16i):
    wid = _wid()
    pltpu.sync_copy(srcR.at[wid], sv)
    pltpu.sync_copy(dstR.at[wid], dv)
    pltpu.sync_copy(wR.at[wid], wv)

    zero16i = jnp.zeros((16,), _i32)
    zero16f = jnp.zeros((16,), _f32)
    iota16 = lax.iota(_i32, 16)

    def zi(b, carry):
        cntv[b, :] = zero16i
        flv[b, :] = zero16i
        return carry

    lax.fori_loop(0, NW, zi, 0)

    def _flush(b, fl):
        base = lax.shift_right_logical(fl, 3) * C3 + (fl & 7) * 16
        pltpu.sync_copy(stS.at[b], pk.at[wid, b, pl.ds(base, 16)])
        pltpu.sync_copy(stD.at[b], pk.at[wid, b, pl.ds(base + CB, 16)])
        pltpu.sync_copy(stW.at[b], pk.at[wid, b, pl.ds(base + 2 * CB, 16)])

    def group(gi, carry):
        s16 = sv[pl.ds(gi * 16, 16)]
        d16 = dv[pl.ds(gi * 16, 16)]
        w16 = wv[pl.ds(gi * 16, 16)]
        b16 = lax.shift_right_logical(d16 * 6554, 21)
        dl16 = d16 - b16 * NB
        w16i = lax.bitcast_convert_type(w16, _i32)
        for l in range(16):
            b = b16[l]
            pos16 = cntv[b, :]
            pos = pos16[0]
            msk = iota16 == jnp.full((16,), pos, _i32)
            stS[b, :] = jnp.where(msk, jnp.full((16,), s16[l], _i32), stS[b, :])
            stD[b, :] = jnp.where(msk, jnp.full((16,), dl16[l], _i32), stD[b, :])
            stW[b, :] = jnp.where(msk, jnp.full((16,), w16i[l], _i32), stW[b, :])

            @pl.when(pos == 15)
            def _():
                _flush(b, flv[b, :][0])
                flv[b, :] = flv[b, :] + 1
                cntv[b, :] = zero16i

            @pl.when(pos < 15)
            def _():
                cntv[b, :] = pos16 + 1
        return carry

    lax.fori_loop(0, EC * CB // 16, group, 0)

    def drain(b, carry):
        pos = cntv[b, :][0]

        @pl.when(pos > 0)
        def _():
            keep = iota16 < jnp.full((16,), pos, _i32)
            stW[b, :] = jnp.where(keep, stW[b, :], zero16i)
            _flush(b, flv[b, :][0])
            flv[b, :] = flv[b, :] + 1

        fl2 = flv[b, :][0]
        npad = jnp.where(fl2 == 0, 8, (8 - lax.rem(fl2, 8)) & 7)
        stS[b, :] = zero16i
        stD[b, :] = zero16i
        stW[b, :] = zero16i

        def padf(i, c2):
            _flush(b, fl2 + i)
            return c2

        lax.fori_loop(0, npad, padf, 0)
        nch = lax.shift_right_logical(fl2 + npad, 3)
        tmp16i[:] = jnp.full((16,), nch, _i32)
        pltpu.sync_copy(tmp16i, cnts.at[b, wid])
        return carry

    lax.fori_loop(0, NW, drain, 0)


_part = pl.kernel(
    _part_body,
    out_type=(
        jax.ShapeDtypeStruct((NW, NW, CAP3), _i32),
        jax.ShapeDtypeStruct((NW, NW, 16), _i32),
    ),
    mesh=_mesh,
    scratch_types=[
        pltpu.VMEM((EC * CB,), _i32),
        pltpu.VMEM((EC * CB,), _i32),
        pltpu.VMEM((EC * CB,), _f32),
        pltpu.VMEM((NW, 16), _i32),
        pltpu.VMEM((NW, 16), _i32),
        pltpu.VMEM((NW, 16), _i32),
        pltpu.VMEM((NW, 16), _i32),
        pltpu.VMEM((NW, 16), _i32),
        pltpu.VMEM((16,), _i32),
    ],
)


def _deg_body(pk, cnts, out, pc, cv, degT):
    oid = _wid()
    zero16 = jnp.zeros((16,), _f32)

    def z(i, carry):
        degT[i, :] = zero16
        return carry

    lax.fori_loop(0, NB, z, 0)
    pltpu.sync_copy(cnts.at[oid], cv)

    def seg(w, carry):
        nch = cv[w, :][0]

        def chunk(j, c2):
            pltpu.sync_copy(pk.at[w, oid, pl.ds(j * C3, C3)], pc)

            def grp(bg, c3):
                d16 = pc[pl.ds(CB + bg * 16, 16)]
                w16 = lax.bitcast_convert_type(pc[pl.ds(2 * CB + bg * 16, 16)], _f32)
                for l in range(16):
                    dloc = d16[l]
                    degT[dloc, :] = degT[dloc, :] + jnp.full((16,), w16[l], _f32)
                return c3

            lax.fori_loop(0, CB // 16, grp, 0)
            return c2

        lax.fori_loop(0, nch, chunk, 0)
        return carry

    lax.fori_loop(0, NW, seg, 0)
    pltpu.sync_copy(degT, out.at[pl.ds(oid * NB, NB)])


_deg = pl.kernel(
    _deg_body,
    out_type=jax.ShapeDtypeStruct((NPAD, 16), _f32),
    mesh=_mesh,
    scratch_types=[
        pltpu.VMEM((C3,), _i32),
        pltpu.VMEM((NW, 16), _i32),
        pltpu.VMEM((NB, 16), _f32),
    ],
)


def _hop_body(g, pk, cnts, out, cv,
              pk0, pk1, pk2, rows0, rows1,
              sS0, sS1, sS2, sG0, sG1, accT):
    oid = _wid()
    zero16 = jnp.zeros((16,), _f32)

    def z(i, carry):
        for gg in range(8):
            accT[i, pl.ds(gg * 16, 16)] = zero16
        return carry

    lax.fori_loop(0, NB, z, 0)
    pltpu.sync_copy(cnts.at[oid], cv)

    def nch_of(w):
        return cv[w, :][0]

    def tot_f(w, t):
        return t + nch_of(w)

    total = lax.fori_loop(0, NW, tot_f, 0)

    def nxt(w, j):
        j2 = j + 1
        wrap = j2 >= nch_of(w)
        return jnp.where(wrap, jnp.minimum(w + 1, NW - 1), w), jnp.where(wrap, 0, j2)

    pks = (pk0, pk1, pk2)
    rws = (rows0, rows1)
    sSs = (sS0, sS1, sS2)
    sGs = (sG0, sG1)

    GS = 4
    GR = CB // GS

    def stage(w, j, pbuf, sem):
        pltpu.async_copy(pk.at[w, oid, pl.ds(j * C3, C3)], pbuf, sem)

    def gather(pbuf, rbuf, sem):
        for k in range(GS):
            pltpu.async_copy(
                g.at[pbuf.at[pl.ds(k * GR, GR)]],
                rbuf.at[pl.ds(k * GR, GR)], sem,
            )

    def wait_stage(pbuf, sem):
        pltpu.make_async_copy(pk.at[0, 0, pl.ds(0, C3)], pbuf, sem).wait()

    def wait_gather(rbuf, sem):
        for k in range(GS):
            pltpu.make_async_copy(
                g.at[pl.ds(0, GR)], rbuf.at[pl.ds(k * GR, GR)], sem,
            ).wait()

    def compute(pbuf, rbuf):
        def grp(bg, c3):
            d16 = pbuf[pl.ds(CB + bg * 16, 16)]
            w16 = lax.bitcast_convert_type(pbuf[pl.ds(2 * CB + bg * 16, 16)], _f32)
            for l in range(16):
                e = bg * 16 + l
                dloc = d16[l]
                n16 = jnp.full((16,), w16[l], _f32)
                acc = [accT[dloc, pl.ds(gg * 16, 16)] for gg in range(8)]
                vals = [acc[gg] + n16 * rbuf[e, pl.ds(gg * 16, 16)]
                        for gg in range(8)]
                for gg in range(8):
                    accT[dloc, pl.ds(gg * 16, 16)] = vals[gg]
            return c3

        lax.fori_loop(0, CB // 16, grp, 0)

    w1, j1 = nxt(0, 0)
    w2, j2 = nxt(w1, j1)
    pltpu.sync_copy(pk.at[0, oid, pl.ds(0, C3)], pk0)

    @pl.when(total > 1)
    def _():
        stage(w1, j1, pk1, sS1)

    gather(pk0, rows0, sG0)

    NT6 = lax.div(total + 5, 6)

    def macro(i6, carry):
        w0c, j0c, w1c, j1c, w2c, j2c = carry
        for u in range(6):
            i = i6 * 6 + u
            pA = pks[u % 3]
            pB = pks[(u + 1) % 3]
            pC = pks[(u + 2) % 3]
            rA = rws[u % 2]
            rB = rws[(u + 1) % 2]

            @pl.when(i + 1 < total)
            def _(pB=pB, rB=rB, u=u, w1c=w1c, j1c=j1c, w2c=w2c, j2c=j2c, i=i):
                wait_stage(pB, sSs[(u + 1) % 3])
                gather(pB, rB, sGs[(u + 1) % 2])

                @pl.when(i + 2 < total)
                def _():
                    stage(w2c, j2c, pks[(u + 2) % 3], sSs[(u + 2) % 3])

            @pl.when(i < total)
            def _(pA=pA, rA=rA, u=u):
                wait_gather(rA, sGs[u % 2])
                compute(pA, rA)

            w0c, j0c = w1c, j1c
            w1c, j1c = w2c, j2c
            w2c, j2c = nxt(w2c, j2c)
        return (w0c, j0c, w1c, j1c, w2c, j2c)

    lax.fori_loop(0, NT6, macro, (0, 0, w1, j1, w2, j2))
    pltpu.sync_copy(accT, out.at[pl.ds(oid * NB, NB)])


_hop = pl.kernel(
    _hop_body,
    out_type=jax.ShapeDtypeStruct((NPAD, F), _f32),
    mesh=_mesh,
    scratch_types=[
        pltpu.VMEM((NW, 16), _i32),
        pltpu.VMEM((C3,), _i32),
        pltpu.VMEM((C3,), _i32),
        pltpu.VMEM((C3,), _i32),
        pltpu.VMEM((CB, F), _f32),
        pltpu.VMEM((CB, F), _f32),
        pltpu.SemaphoreType.DMA,
        pltpu.SemaphoreType.DMA,
        pltpu.SemaphoreType.DMA,
        pltpu.SemaphoreType.DMA,
        pltpu.SemaphoreType.DMA,
        pltpu.VMEM((NB, F), _f32),
    ],
)


_row = pl.BlockSpec((R, F), lambda i: (i, 0))
_colb = pl.BlockSpec((R, 1), lambda i: (i, 0))


def _full(shp):
    return pl.BlockSpec(shp, lambda i: tuple(0 for _ in shp))


def _dis_body(deg_ref, dis_ref):
    dsum = deg_ref[...]
    dis = jnp.where(dsum > 0, lax.rsqrt(dsum), 0.0)
    dis_ref[...] = dis[:, 0:1]


_dis = pl.pallas_call(
    _dis_body,
    out_shape=jax.ShapeDtypeStruct((NPAD, 1), _f32),
)


def _enc_body(x_ref, W_ref, b_ref, gl_ref, dis_ref, g_ref, acc_ref):
    x = x_ref[...]
    t = jnp.dot(x, W_ref[0], preferred_element_type=_f32) + b_ref[0]
    h1 = _leaky(t)
    t2 = jnp.dot(h1, W_ref[1], preferred_element_type=_f32) + b_ref[1]
    h = _leaky(t2)
    g_ref[...] = dis_ref[...] * h
    acc_ref[...] = jnp.dot(h, gl_ref[...], preferred_element_type=_f32)


_enc = pl.pallas_call(
    _enc_body,
    grid=(GRID,),
    in_specs=[_row, _full((2, F, F)), _full((2, F)), _full((F, F)), _colb],
    out_specs=[_row, _row],
    out_shape=[
        jax.ShapeDtypeStruct((NPAD, F), _f32),
        jax.ShapeDtypeStruct((NPAD, F), _f32),
    ],
)


def _step_body(p_ref, accin_ref, W_ref, dis_ref, g_ref, acc_ref):
    dis = dis_ref[...]
    xk = dis * p_ref[...]
    g_ref[...] = dis * xk
    acc_ref[...] = accin_ref[...] + jnp.dot(
        xk, W_ref[...], preferred_element_type=_f32
    )


_step = pl.pallas_call(
    _step_body,
    grid=(GRID,),
    in_specs=[_row, _row, _full((F, F)), _colb],
    out_specs=[_row, _row],
    out_shape=[
        jax.ShapeDtypeStruct((NPAD, F), _f32),
        jax.ShapeDtypeStruct((NPAD, F), _f32),
    ],
)


def _lend_body(p_ref, accin_ref, W_ref, b_ref, dis_ref, Wn_ref, g_ref, accn_ref):
    dis = dis_ref[...]
    xk = dis * p_ref[...]
    t = accin_ref[...] + jnp.dot(xk, W_ref[...], preferred_element_type=_f32)
    t = t + b_ref[...]
    h = _leaky(t)
    g_ref[...] = dis * h
    accn_ref[...] = jnp.dot(h, Wn_ref[...], preferred_element_type=_f32)


_lend = pl.pallas_call(
    _lend_body,
    grid=(GRID,),
    in_specs=[_row, _row, _full((F, F)), _full((1, F)), _colb, _full((F, F))],
    out_specs=[_row, _row],
    out_shape=[
        jax.ShapeDtypeStruct((NPAD, F), _f32),
        jax.ShapeDtypeStruct((NPAD, F), _f32),
    ],
)


def _fin_body(p_ref, accin_ref, W_ref, b_ref, dis_ref, dW_ref, db_ref, out_ref):
    dis = dis_ref[...]
    xk = dis * p_ref[...]
    t = accin_ref[...] + jnp.dot(xk, W_ref[...], preferred_element_type=_f32)
    t = t + b_ref[...]
    d = _leaky(jnp.dot(t, dW_ref[0], preferred_element_type=_f32) + db_ref[0])
    out_ref[...] = jnp.dot(d, dW_ref[1], preferred_element_type=_f32) + db_ref[1]


_fin = pl.pallas_call(
    _fin_body,
    grid=(GRID,),
    in_specs=[_row, _row, _full((F, F)), _full((1, F)), _colb,
              _full((2, F, F)), _full((2, F))],
    out_specs=_row,
    out_shape=jax.ShapeDtypeStruct((NPAD, F), _f32),
)


def kernel(x, edge_index, edge_weight, enc_W, enc_b, gnn_lins, gnn_bias, dec_W, dec_b):
    ei = edge_index.astype(_i32)
    src = jnp.pad(ei[0], (0, EP - E)).reshape(NW, EC * CB)
    dst = jnp.pad(ei[1], (0, EP - E)).reshape(NW, EC * CB)
    wR = jnp.pad(edge_weight, (0, EP - E)).reshape(NW, EC * CB)
    x_p = jnp.pad(x, ((0, NPAD - N), (0, 0)))

    pkb, cnts = _part(src, dst, wR)
    deg = _deg(pkb, cnts)
    dis_col = _dis(deg)
    g, acc = _enc(x_p, enc_W, enc_b, gnn_lins[0, 0], dis_col)

    for l in range(2):
        for k in range(1, 4):
            p = _hop(g, pkb, cnts)
            if k < 3:
                g, acc = _step(p, acc, gnn_lins[l, k], dis_col)
        if l == 0:
            g, acc = _lend(
                p, acc, gnn_lins[0, 3],
                gnn_bias[0].reshape(1, F), dis_col, gnn_lins[1, 0],
            )
        else:
            out = _fin(
                p, acc, gnn_lins[1, 3],
                gnn_bias[1].reshape(1, F), dis_col, dec_W, dec_b,
            )
    return out[:N]
